# baseline (speedup 1.0000x reference)
import functools

import jax
import jax.numpy as jnp
from jax import lax
from jax.experimental import pallas as pl
from jax.experimental.pallas import tpu as pltpu
from jax.experimental.pallas import tpu_sc as plsc

NUM_USERS = 100000
NUM_ITEMS = 100000
EMBED_DIM = 64
BATCH = 16384

_NC = 2
_NS = 16
_NW = _NC * _NS
_BPW = BATCH // _NW
_GROUPS = _BPW // 16


def _sc_body(uid_hbm, iid_hbm, ue_hbm, ie_hbm, ug_hbm, ig_hbm,
             dot_hbm, s_hbm,
             uid_v, iid_v, ue_v, ie_v, ug_v, ig_v, dot_v, s_v,
             sem_ue, sem_ie, sem_ug, sem_ig):
    wid = lax.axis_index("s") * _NC + lax.axis_index("c")
    base = wid * _BPW

    pltpu.sync_copy(uid_hbm.at[pl.ds(base, _BPW)], uid_v)
    pltpu.sync_copy(iid_hbm.at[pl.ds(base, _BPW)], iid_v)
    cp_ue = pltpu.async_copy(ue_hbm.at[uid_v], ue_v, sem_ue)
    cp_ie = pltpu.async_copy(ie_hbm.at[iid_v], ie_v, sem_ie)
    cp_ug = pltpu.async_copy(ug_hbm.at[uid_v], ug_v, sem_ug)
    cp_ig = pltpu.async_copy(ig_hbm.at[iid_v], ig_v, sem_ig)
    cp_ue.wait()
    cp_ie.wait()
    cp_ug.wait()
    cp_ig.wait()

    lane = jnp.arange(16, dtype=jnp.int32)

    def group(g, _):
        r0 = g * 16
        row = lane + r0
        acc = jnp.zeros((16,), jnp.float32)
        for j in range(EMBED_DIM):
            c = (lane + j) & (EMBED_DIM - 1)
            u = plsc.load_gather(ue_v, [row, c])
            v = plsc.load_gather(ie_v, [row, c])
            acc = acc + u * v
        dot_v[pl.ds(r0, 16)] = acc
        s_v[pl.ds(r0, 16)] = ug_v[pl.ds(r0, 16)] + ig_v[pl.ds(r0, 16)]
        return _

    lax.fori_loop(0, _GROUPS, group, None)

    pltpu.sync_copy(dot_v, dot_hbm.at[pl.ds(base, _BPW)])
    pltpu.sync_copy(s_v, s_hbm.at[pl.ds(base, _BPW)])


@jax.jit
def _sc_call(uid, iid, ue, ie, ug, ig):
    mesh = plsc.VectorSubcoreMesh(core_axis_name="c", subcore_axis_name="s")
    f = functools.partial(
        pl.kernel, _sc_body, mesh=mesh,
        compiler_params=pltpu.CompilerParams(
            needs_layout_passes=False, use_tc_tiling_on_sc=False),
        out_type=[
            jax.ShapeDtypeStruct((BATCH,), jnp.float32),
            jax.ShapeDtypeStruct((BATCH,), jnp.float32),
        ],
        scratch_types=[
            pltpu.VMEM((_BPW,), jnp.int32),
            pltpu.VMEM((_BPW,), jnp.int32),
            pltpu.VMEM((_BPW, EMBED_DIM), jnp.float32),
            pltpu.VMEM((_BPW, EMBED_DIM), jnp.float32),
            pltpu.VMEM((_BPW,), jnp.float32),
            pltpu.VMEM((_BPW,), jnp.float32),
            pltpu.VMEM((_BPW,), jnp.float32),
            pltpu.VMEM((_BPW,), jnp.float32),
            pltpu.SemaphoreType.DMA,
            pltpu.SemaphoreType.DMA,
            pltpu.SemaphoreType.DMA,
            pltpu.SemaphoreType.DMA,
        ],
    )()
    return f(uid, iid, ue, ie, ug, ig)


def _tc_softplus_body(s_ref, o_ref):
    o_ref[...] = jax.nn.softplus(s_ref[...])


@jax.jit
def _tc_softplus(s2d):
    return pl.pallas_call(
        _tc_softplus_body,
        out_shape=jax.ShapeDtypeStruct(s2d.shape, s2d.dtype),
    )(s2d)


def kernel(user_ids, item_ids, user_emb, item_emb, user_gamma, item_gamma):
    uid = user_ids.astype(jnp.int32)
    iid = item_ids.astype(jnp.int32)
    ug1 = user_gamma.reshape(NUM_USERS)
    ig1 = item_gamma.reshape(NUM_ITEMS)
    dot, s = _sc_call(uid, iid, user_emb, item_emb, ug1, ig1)
    var = _tc_softplus(s.reshape(128, 128)).reshape(BATCH)
    return (dot, var)
